# baseline (speedup 1.0000x reference)
import functools

import jax
import jax.numpy as jnp
from jax import lax
from jax.experimental import pallas as pl
from jax.experimental.pallas import tpu as pltpu
from jax.experimental.pallas import tpu_sc as plsc

_NC = 2
_NS = 16
_NW = _NC * _NS
_LANES = 16
_TBC = 4096


def _embedding_bag_sum(x, table):
    B, H = x.shape
    _, D = table.shape
    BPW = B // _NW
    CB = 2
    ROWS = CB * H
    NCH = BPW // CB
    NBUF = 4

    xw = x.reshape(_NW, NCH, ROWS)

    mesh = plsc.VectorSubcoreMesh(core_axis_name="c", subcore_axis_name="s")

    @functools.partial(
        pl.kernel,
        mesh=mesh,
        compiler_params=pltpu.CompilerParams(use_tc_tiling_on_sc=False),
        out_type=jax.ShapeDtypeStruct((B, D), jnp.float32),
        scratch_types=[
            pltpu.VMEM((NCH, ROWS), jnp.int32),
            pltpu.VMEM((NBUF, ROWS, D), jnp.float32),
            pltpu.VMEM((BPW, D), jnp.float32),
            pltpu.SemaphoreType.DMA,
            pltpu.SemaphoreType.DMA((NBUF,)),
        ],
    )
    def ebag(x_hbm, table_hbm, out_hbm, idx_v, rows_v, out_v, sem, gsems):
        wid = lax.axis_index("s") * _NC + lax.axis_index("c")
        pltpu.async_copy(x_hbm.at[wid], idx_v, sem).wait()

        def gather_start(c, b):
            pltpu.async_copy(
                table_hbm.at[idx_v.at[c]], rows_v.at[b], gsems.at[b]
            )

        for b in range(NBUF):
            gather_start(b, b)

        @pl.loop(0, NCH, step=NBUF)
        def _(c0):
            for b in range(NBUF):
                c = c0 + b
                pltpu.make_async_copy(
                    table_hbm.at[idx_v.at[c]], rows_v.at[b], gsems.at[b]
                ).wait()

                rows_b = rows_v.at[b]
                G = 10
                for bag in range(CB):
                    for k in range(D // _LANES):
                        col = pl.ds(k * _LANES, _LANES)
                        acc = None
                        for g0 in range(0, H, G):
                            vals = [
                                rows_b[bag * H + r, col]
                                for r in range(g0, min(g0 + G, H))
                            ]
                            while len(vals) > 1:
                                vals = [
                                    vals[i] + vals[i + 1]
                                    for i in range(0, len(vals) - 1, 2)
                                ] + ([vals[-1]] if len(vals) % 2 else [])
                            acc = vals[0] if acc is None else acc + vals[0]
                        out_v[c * CB + bag, col] = acc

                @pl.when(c + NBUF < NCH)
                def _():
                    gather_start(c + NBUF, b)

        pltpu.async_copy(out_v, out_hbm.at[pl.ds(wid * BPW, BPW)], sem).wait()

    return ebag(xw, table)


def _linearize_table(table):
    V, D = table.shape
    BC = _TBC
    grid = -(-V // (2 * BC))
    V2 = grid * 2 * BC
    tableT = table.T

    def body(t_ref, o_ref):
        for t in range(BC // 128):
            a = t_ref[:, pl.ds(t * 128, 128)][...].T
            b = t_ref[:, pl.ds(BC + t * 128, 128)][...].T
            cat = jnp.concatenate([a, b], axis=1)
            o_ref[pl.ds(t * 128 * 2 * D, 128 * 2 * D)] = cat.reshape(
                128 * 2 * D
            )

    out = pl.pallas_call(
        body,
        grid=(grid,),
        in_specs=[pl.BlockSpec((D, 2 * BC), lambda c: (0, c))],
        out_specs=pl.BlockSpec((BC * 2 * D,), lambda c: (c,)),
        out_shape=jax.ShapeDtypeStruct((V2 * D,), jnp.float32),
        compiler_params=pltpu.CompilerParams(
            dimension_semantics=("parallel",)
        ),
    )(tableT)
    return out, V2


def _mlp(pooled_sum, W1, b1, W2, b2, hist):
    B, D = pooled_sum.shape
    HN = W1.shape[0]
    O = W2.shape[0]
    BM = 1024
    inv = 1.0 / float(hist)

    def body(p_ref, w1_ref, b1_ref, w2_ref, b2_ref, o_ref):
        p = p_ref[...] * inv
        h = jnp.dot(p, w1_ref[...], preferred_element_type=jnp.float32)
        h = jnp.maximum(h + b1_ref[...], 0.0)
        o = jnp.sum(h * w2_ref[...], axis=1, keepdims=True) + b2_ref[...]
        o_ref[...] = jax.nn.sigmoid(o)

    return pl.pallas_call(
        body,
        grid=(B // BM,),
        in_specs=[
            pl.BlockSpec((BM, D), lambda i: (i, 0)),
            pl.BlockSpec((D, HN), lambda i: (0, 0)),
            pl.BlockSpec((1, HN), lambda i: (0, 0)),
            pl.BlockSpec((1, HN), lambda i: (0, 0)),
            pl.BlockSpec((1, O), lambda i: (0, 0)),
        ],
        out_specs=pl.BlockSpec((BM, O), lambda i: (i, 0)),
        out_shape=jax.ShapeDtypeStruct((B, O), jnp.float32),
    )(pooled_sum, W1.T, b1.reshape(1, HN), W2.reshape(1, HN), b2.reshape(1, O))


def kernel(x, table, W1, b1, W2, b2):
    V, D = table.shape
    BC = _TBC
    out1d, V2 = _linearize_table(table)
    table_lin = out1d.reshape(V2, D)
    g = x // (2 * BC)
    rem = x % (2 * BC)
    slot = g * BC + (rem % BC)
    x2 = 2 * slot + (rem // BC)
    pooled_sum = _embedding_bag_sum(x2, table_lin)
    return _mlp(pooled_sum, W1, b1, W2, b2, x.shape[1])


# speedup vs baseline: 1.1040x; 1.1040x over previous
import functools

import jax
import jax.numpy as jnp
from jax import lax
from jax.experimental import pallas as pl
from jax.experimental.pallas import tpu as pltpu
from jax.experimental.pallas import tpu_sc as plsc

_NC = 2
_NS = 16
_NW = _NC * _NS
_LANES = 16
_TBC = 16384


def _embedding_bag_sum(x, table):
    B, H = x.shape
    _, D = table.shape
    BPW = B // _NW
    CB = 2
    ROWS = CB * H
    NCH = BPW // CB
    NBUF = 4

    xw = x.reshape(_NW, NCH, ROWS)

    mesh = plsc.VectorSubcoreMesh(core_axis_name="c", subcore_axis_name="s")

    @functools.partial(
        pl.kernel,
        mesh=mesh,
        compiler_params=pltpu.CompilerParams(use_tc_tiling_on_sc=False),
        out_type=jax.ShapeDtypeStruct((B, D), jnp.float32),
        scratch_types=[
            pltpu.VMEM((NCH, ROWS), jnp.int32),
            pltpu.VMEM((NBUF, ROWS, D), jnp.float32),
            pltpu.VMEM((BPW, D), jnp.float32),
            pltpu.SemaphoreType.DMA,
            pltpu.SemaphoreType.DMA((NBUF,)),
        ],
    )
    def ebag(x_hbm, table_hbm, out_hbm, idx_v, rows_v, out_v, sem, gsems):
        wid = lax.axis_index("s") * _NC + lax.axis_index("c")
        pltpu.async_copy(x_hbm.at[wid], idx_v, sem).wait()

        def gather_start(c, b):
            pltpu.async_copy(
                table_hbm.at[idx_v.at[c]], rows_v.at[b], gsems.at[b]
            )

        for b in range(NBUF):
            gather_start(b, b)

        @pl.loop(0, NCH, step=NBUF)
        def _(c0):
            for b in range(NBUF):
                c = c0 + b
                pltpu.make_async_copy(
                    table_hbm.at[idx_v.at[c]], rows_v.at[b], gsems.at[b]
                ).wait()

                rows_b = rows_v.at[b]
                G = 10
                for bag in range(CB):
                    for k in range(D // _LANES):
                        col = pl.ds(k * _LANES, _LANES)
                        acc = None
                        for g0 in range(0, H, G):
                            vals = [
                                rows_b[bag * H + r, col]
                                for r in range(g0, min(g0 + G, H))
                            ]
                            while len(vals) > 1:
                                vals = [
                                    vals[i] + vals[i + 1]
                                    for i in range(0, len(vals) - 1, 2)
                                ] + ([vals[-1]] if len(vals) % 2 else [])
                            acc = vals[0] if acc is None else acc + vals[0]
                        out_v[c * CB + bag, col] = acc

                @pl.when(c + NBUF < NCH)
                def _():
                    gather_start(c + NBUF, b)

        pltpu.async_copy(out_v, out_hbm.at[pl.ds(wid * BPW, BPW)], sem).wait()

    return ebag(xw, table)


def _linearize_table(table):
    V, D = table.shape
    BC = _TBC
    grid = -(-V // (2 * BC))
    V2 = grid * 2 * BC
    tableT = table.T

    def body(t_ref, o_ref):
        for t in range(BC // 128):
            a = t_ref[:, pl.ds(t * 128, 128)][...].T
            b = t_ref[:, pl.ds(BC + t * 128, 128)][...].T
            cat = jnp.concatenate([a, b], axis=1)
            o_ref[pl.ds(t * 128 * 2 * D, 128 * 2 * D)] = cat.reshape(
                128 * 2 * D
            )

    out = pl.pallas_call(
        body,
        grid=(grid,),
        in_specs=[pl.BlockSpec((D, 2 * BC), lambda c: (0, c))],
        out_specs=pl.BlockSpec((BC * 2 * D,), lambda c: (c,)),
        out_shape=jax.ShapeDtypeStruct((V2 * D,), jnp.float32),
        compiler_params=pltpu.CompilerParams(
            dimension_semantics=("parallel",)
        ),
    )(tableT)
    return out, V2


def _mlp(pooled_sum, W1, b1, W2, b2, hist):
    B, D = pooled_sum.shape
    HN = W1.shape[0]
    O = W2.shape[0]
    BM = 1024
    inv = 1.0 / float(hist)

    def body(p_ref, w1_ref, b1_ref, w2_ref, b2_ref, o_ref):
        p = p_ref[...] * inv
        h = jnp.dot(p, w1_ref[...], preferred_element_type=jnp.float32)
        h = jnp.maximum(h + b1_ref[...], 0.0)
        o = jnp.sum(h * w2_ref[...], axis=1, keepdims=True) + b2_ref[...]
        o_ref[...] = jax.nn.sigmoid(o)

    return pl.pallas_call(
        body,
        grid=(B // BM,),
        in_specs=[
            pl.BlockSpec((BM, D), lambda i: (i, 0)),
            pl.BlockSpec((D, HN), lambda i: (0, 0)),
            pl.BlockSpec((1, HN), lambda i: (0, 0)),
            pl.BlockSpec((1, HN), lambda i: (0, 0)),
            pl.BlockSpec((1, O), lambda i: (0, 0)),
        ],
        out_specs=pl.BlockSpec((BM, O), lambda i: (i, 0)),
        out_shape=jax.ShapeDtypeStruct((B, O), jnp.float32),
    )(pooled_sum, W1.T, b1.reshape(1, HN), W2.reshape(1, HN), b2.reshape(1, O))


def kernel(x, table, W1, b1, W2, b2):
    V, D = table.shape
    BC = _TBC
    out1d, V2 = _linearize_table(table)
    table_lin = out1d.reshape(V2, D)
    g = x // (2 * BC)
    rem = x % (2 * BC)
    slot = g * BC + (rem % BC)
    x2 = 2 * slot + (rem // BC)
    pooled_sum = _embedding_bag_sum(x2, table_lin)
    return _mlp(pooled_sum, W1, b1, W2, b2, x.shape[1])
